# baseline (speedup 1.0000x reference)
import functools

import jax
import jax.numpy as jnp
from jax import lax
from jax.experimental import pallas as pl
from jax.experimental.pallas import tpu as pltpu
from jax.experimental.pallas import tpu_sc as plsc

B, N, D = 16, 4096, 256
ROWS = B * N
NW = 32
RPW = ROWS // NW
VPR = N // 16
VPW = RPW // 16
CHUNK = 128
NCHUNK = RPW // CHUNK
NB = 3
SCAN_PER_CHUNK = VPR // NCHUNK


@functools.cache
def _make_sc_kernel():
    mesh = plsc.VectorSubcoreMesh(core_axis_name="c", subcore_axis_name="s")

    @functools.partial(
        pl.kernel,
        mesh=mesh,
        out_type=jax.ShapeDtypeStruct((ROWS, D), jnp.float32),
        scratch_types=[
            pltpu.VMEM((N,), jnp.float32),
            pltpu.VMEM((N,), jnp.int32),
            pltpu.VMEM((RPW + 16,), jnp.int32),
            pltpu.VMEM((RPW + 16,), jnp.int32),
            pltpu.VMEM((CHUNK, D), jnp.float32),
            pltpu.VMEM((CHUNK, D), jnp.float32),
            pltpu.VMEM((CHUNK, D), jnp.float32),
            pltpu.VMEM((16, D), jnp.float32),
            pltpu.SemaphoreType.DMA,
            pltpu.SemaphoreType.DMA,
            pltpu.SemaphoreType.DMA,
            pltpu.SemaphoreType.DMA,
            pltpu.SemaphoreType.DMA,
            pltpu.SemaphoreType.DMA,
            pltpu.SemaphoreType.DMA,
            pltpu.SemaphoreType.DMA,
        ],
    )
    def sc_kernel(x_hbm, mask_hbm, out_hbm, mbuf, ffbuf, csrc, cdst,
                  b0, b1, b2, fbuf, g0, g1, g2, w0, w1, w2, fg, fs):
        w = lax.axis_index("s") * 2 + lax.axis_index("c")
        b = w >> 1
        h = w & 1
        p0 = w * RPW
        pltpu.sync_copy(mask_hbm.at[pl.ds(b * N, N)], mbuf)
        lanes = lax.broadcasted_iota(jnp.int32, (16,), 0)
        fifteen = jnp.full((16,), 15, jnp.int32)
        neg1 = jnp.full((16,), -1, jnp.int32)

        def scan_body(i, carry):
            mv = mbuf[pl.ds(i * 16, 16)]
            pos = i * 16 + lanes
            valid = jnp.where(mv > 0.9, neg1, pos)
            cm = valid
            for s in (1, 2, 4, 8):
                sh = cm.at[jnp.maximum(lanes - s, 0)].get(
                    mode="promise_in_bounds")
                cm = jnp.maximum(cm, jnp.where(lanes >= s, sh, neg1))
            cm = jnp.maximum(cm, carry)
            ffbuf[pl.ds(i * 16, 16)] = cm
            return cm.at[fifteen].get(mode="promise_in_bounds")

        bufs = (b0, b1, b2)
        gsems = (g0, g1, g2)
        wsems = (w0, w1, w2)
        gcp = [None] * NCHUNK
        wcp = [None] * NCHUNK
        for c in range(NB):
            gcp[c] = pltpu.async_copy(
                x_hbm.at[pl.ds(p0 + c * CHUNK, CHUNK)], bufs[c], gsems[c])
        carry = neg1
        for c in range(NCHUNK):
            carry = lax.fori_loop(c * SCAN_PER_CHUNK,
                                  (c + 1) * SCAN_PER_CHUNK,
                                  scan_body, carry)
            k = c % NB
            gcp[c].wait()
            wcp[c] = pltpu.async_copy(
                bufs[k], out_hbm.at[pl.ds(p0 + c * CHUNK, CHUNK)], wsems[k])
            nxt = c + NB
            if nxt < NCHUNK:
                wcp[c].wait()
                gcp[nxt] = pltpu.async_copy(
                    x_hbm.at[pl.ds(p0 + nxt * CHUNK, CHUNK)],
                    bufs[k], gsems[k])
        row_last = carry
        fallback = jnp.maximum(row_last, 0)
        rowbase = b * N

        def comp_body(i, cv):
            v = ffbuf[pl.ds((h * VPW + i) * 16, 16)]
            gidx = rowbase + jnp.where(v >= 0, v, fallback)
            gpos = p0 + i * 16 + lanes
            m = gidx != gpos
            mi = jnp.where(m, jnp.int32(1), jnp.int32(0))
            pre = mi
            for s in (1, 2, 4, 8):
                sh = pre.at[jnp.maximum(lanes - s, 0)].get(
                    mode="promise_in_bounds")
                pre = pre + jnp.where(lanes >= s, sh, jnp.int32(0))
            target = lanes + 1
            lo = jnp.zeros((16,), jnp.int32)
            for s in (8, 4, 2, 1):
                pv = pre.at[lo + (s - 1)].get(mode="promise_in_bounds")
                lo = lo + jnp.where(pv < target, jnp.int32(s), jnp.int32(0))
            lo = jnp.minimum(lo, 15)
            gc = gidx.at[lo].get(mode="promise_in_bounds")
            base = cv[0]
            csrc[pl.ds(base, 16)] = gc
            cdst[pl.ds(base, 16)] = p0 + i * 16 + lo
            return cv + pre.at[fifteen].get(mode="promise_in_bounds")

        cv = lax.fori_loop(0, VPW, comp_body, lanes)
        cnt = cv[0]
        kpad = (cnt + 15) & -16
        jmax = jnp.maximum(cnt - 1, 0)
        vb = jmax & -16
        sv_t = csrc[pl.ds(vb, 16)]
        dv_t = cdst[pl.ds(vb, 16)]
        jl = jnp.full((16,), jmax & 15, jnp.int32)
        csrc[pl.ds(cnt, 16)] = sv_t.at[jl].get(mode="promise_in_bounds")
        cdst[pl.ds(cnt, 16)] = dv_t.at[jl].get(mode="promise_in_bounds")

        for c in range(NCHUNK - NB, NCHUNK):
            wcp[c].wait()

        def fix_body(j, carry2):
            sv = csrc[pl.ds(j * 16, 16)]
            dv = cdst[pl.ds(j * 16, 16)]
            pltpu.async_copy(x_hbm.at[sv], fbuf, fg).wait()
            pltpu.async_copy(fbuf, out_hbm.at[dv], fs).wait()
            return carry2

        lax.fori_loop(0, kpad >> 4, fix_body, jnp.int32(0))

    return sc_kernel


def kernel(input, mask):
    x2d = input.reshape(ROWS, D)
    out = _make_sc_kernel()(x2d, mask.reshape(ROWS))
    return out.reshape(B, N, D)


# speedup vs baseline: 1.0272x; 1.0272x over previous
import functools

import jax
import jax.numpy as jnp
from jax import lax
from jax.experimental import pallas as pl
from jax.experimental.pallas import tpu as pltpu
from jax.experimental.pallas import tpu_sc as plsc

B, N, D = 16, 4096, 256
ROWS = B * N
NW = 32
RPW = ROWS // NW
VPR = N // 16
VPW = RPW // 16
CHUNK = 128
NCHUNK = RPW // CHUNK
NB = 3
SCAN_PER_CHUNK = VPR // NCHUNK


@functools.cache
def _make_sc_kernel():
    mesh = plsc.VectorSubcoreMesh(core_axis_name="c", subcore_axis_name="s")

    @functools.partial(
        pl.kernel,
        mesh=mesh,
        out_type=jax.ShapeDtypeStruct((ROWS, D), jnp.float32),
        scratch_types=[
            pltpu.VMEM((N,), jnp.float32),
            pltpu.VMEM((N,), jnp.int32),
            pltpu.VMEM((RPW + 16,), jnp.int32),
            pltpu.VMEM((RPW + 16,), jnp.int32),
            pltpu.VMEM((CHUNK, D), jnp.float32),
            pltpu.VMEM((CHUNK, D), jnp.float32),
            pltpu.VMEM((CHUNK, D), jnp.float32),
            pltpu.VMEM((32, D), jnp.float32),
            pltpu.SemaphoreType.DMA,
            pltpu.SemaphoreType.DMA,
            pltpu.SemaphoreType.DMA,
            pltpu.SemaphoreType.DMA,
            pltpu.SemaphoreType.DMA,
            pltpu.SemaphoreType.DMA,
            pltpu.SemaphoreType.DMA((2,)),
            pltpu.SemaphoreType.DMA((2,)),
        ],
    )
    def sc_kernel(x_hbm, mask_hbm, out_hbm, mbuf, ffbuf, csrc, cdst,
                  b0, b1, b2, fbuf, g0, g1, g2, w0, w1, w2, fg, fs):
        w = lax.axis_index("s") * 2 + lax.axis_index("c")
        b = w >> 1
        h = w & 1
        p0 = w * RPW
        pltpu.sync_copy(mask_hbm.at[pl.ds(b * N, N)], mbuf)
        lanes = lax.broadcasted_iota(jnp.int32, (16,), 0)
        fifteen = jnp.full((16,), 15, jnp.int32)
        neg1 = jnp.full((16,), -1, jnp.int32)

        def scan_body(i, carry):
            mv = mbuf[pl.ds(i * 16, 16)]
            pos = i * 16 + lanes
            valid = jnp.where(mv > 0.9, neg1, pos)
            cm = valid
            for s in (1, 2, 4, 8):
                sh = cm.at[jnp.maximum(lanes - s, 0)].get(
                    mode="promise_in_bounds")
                cm = jnp.maximum(cm, jnp.where(lanes >= s, sh, neg1))
            cm = jnp.maximum(cm, carry)
            ffbuf[pl.ds(i * 16, 16)] = cm
            return cm.at[fifteen].get(mode="promise_in_bounds")

        bufs = (b0, b1, b2)
        gsems = (g0, g1, g2)
        wsems = (w0, w1, w2)
        gcp = [None] * NCHUNK
        wcp = [None] * NCHUNK
        for c in range(NB):
            gcp[c] = pltpu.async_copy(
                x_hbm.at[pl.ds(p0 + c * CHUNK, CHUNK)], bufs[c], gsems[c])
        carry = neg1
        for c in range(NCHUNK):
            carry = lax.fori_loop(c * SCAN_PER_CHUNK,
                                  (c + 1) * SCAN_PER_CHUNK,
                                  scan_body, carry)
            k = c % NB
            gcp[c].wait()
            wcp[c] = pltpu.async_copy(
                bufs[k], out_hbm.at[pl.ds(p0 + c * CHUNK, CHUNK)], wsems[k])
            nxt = c + NB
            if nxt < NCHUNK:
                wcp[c].wait()
                gcp[nxt] = pltpu.async_copy(
                    x_hbm.at[pl.ds(p0 + nxt * CHUNK, CHUNK)],
                    bufs[k], gsems[k])
        row_last = carry
        fallback = jnp.maximum(row_last, 0)
        rowbase = b * N

        def comp_body(i, cv):
            v = ffbuf[pl.ds((h * VPW + i) * 16, 16)]
            gidx = rowbase + jnp.where(v >= 0, v, fallback)
            gpos = p0 + i * 16 + lanes
            m = gidx != gpos
            mi = jnp.where(m, jnp.int32(1), jnp.int32(0))
            pre = mi
            for s in (1, 2, 4, 8):
                sh = pre.at[jnp.maximum(lanes - s, 0)].get(
                    mode="promise_in_bounds")
                pre = pre + jnp.where(lanes >= s, sh, jnp.int32(0))
            target = lanes + 1
            lo = jnp.zeros((16,), jnp.int32)
            for s in (8, 4, 2, 1):
                pv = pre.at[lo + (s - 1)].get(mode="promise_in_bounds")
                lo = lo + jnp.where(pv < target, jnp.int32(s), jnp.int32(0))
            lo = jnp.minimum(lo, 15)
            gc = gidx.at[lo].get(mode="promise_in_bounds")
            base = cv[0]
            csrc[pl.ds(base, 16)] = gc
            cdst[pl.ds(base, 16)] = p0 + i * 16 + lo
            return cv + pre.at[fifteen].get(mode="promise_in_bounds")

        cv = lax.fori_loop(0, VPW, comp_body, lanes)
        cnt = cv[0]
        kpad = (cnt + 15) & -16
        jmax = jnp.maximum(cnt - 1, 0)
        vb = jmax & -16
        sv_t = csrc[pl.ds(vb, 16)]
        dv_t = cdst[pl.ds(vb, 16)]
        jl = jnp.full((16,), jmax & 15, jnp.int32)
        csrc[pl.ds(cnt, 16)] = sv_t.at[jl].get(mode="promise_in_bounds")
        cdst[pl.ds(cnt, 16)] = dv_t.at[jl].get(mode="promise_in_bounds")

        for c in range(NCHUNK - NB, NCHUNK):
            wcp[c].wait()

        nj = kpad >> 4

        @pl.when(nj > 0)
        def _():
            sv0 = csrc[pl.ds(0, 16)]
            pltpu.async_copy(
                x_hbm.at[sv0], fbuf.at[pl.ds(0, 16)], fg.at[0])

        def fix_body(j, carry2):
            sv = csrc[pl.ds(j * 16, 16)]
            dv = cdst[pl.ds(j * 16, 16)]
            slot = fbuf.at[pl.ds((j & 1) * 16, 16)]
            other = fbuf.at[pl.ds(((j + 1) & 1) * 16, 16)]
            pltpu.make_async_copy(x_hbm.at[sv], slot, fg.at[j & 1]).wait()

            @pl.when(j >= 1)
            def _():
                pltpu.make_async_copy(
                    other, out_hbm.at[dv], fs.at[(j + 1) & 1]).wait()

            @pl.when(j + 1 < nj)
            def _():
                svn = csrc[pl.ds((j + 1) * 16, 16)]
                pltpu.async_copy(x_hbm.at[svn], other, fg.at[(j + 1) & 1])

            pltpu.async_copy(slot, out_hbm.at[dv], fs.at[j & 1])
            return carry2

        lax.fori_loop(0, nj, fix_body, jnp.int32(0))

        @pl.when(nj > 0)
        def _():
            jl2 = nj - 1
            dvl = cdst[pl.ds(jl2 * 16, 16)]
            pltpu.make_async_copy(
                fbuf.at[pl.ds((jl2 & 1) * 16, 16)],
                out_hbm.at[dvl], fs.at[jl2 & 1]).wait()

    return sc_kernel


def kernel(input, mask):
    x2d = input.reshape(ROWS, D)
    out = _make_sc_kernel()(x2d, mask.reshape(ROWS))
    return out.reshape(B, N, D)


# speedup vs baseline: 1.2719x; 1.2383x over previous
import functools

import jax
import jax.numpy as jnp
from jax import lax
from jax.experimental import pallas as pl
from jax.experimental.pallas import tpu as pltpu
from jax.experimental.pallas import tpu_sc as plsc

B, N, D = 16, 4096, 256
ROWS = B * N
NW = 32
ROWS_PER_W = ROWS // NW
CHUNK = 128
NCHUNK = ROWS_PER_W // CHUNK


def _fill_index_body(mask_ref, gidx_ref):
    m = mask_ref[...]
    pos = lax.broadcasted_iota(jnp.int32, (B, N), 1)
    valid = jnp.where(m > 0.9, jnp.int32(-1), pos)
    ff = valid
    s = 1
    while s < N:
        shifted = jnp.concatenate(
            [jnp.full((B, s), -1, jnp.int32), ff[:, : N - s]], axis=1)
        ff = jnp.maximum(ff, shifted)
        s *= 2
    last = jnp.max(valid, axis=1, keepdims=True)
    fill = jnp.where(ff >= 0, ff, jnp.broadcast_to(last, (B, N)))
    fill = jnp.maximum(fill, 0)
    row = lax.broadcasted_iota(jnp.int32, (B, N), 0)
    gidx_ref[...] = fill + row * N


_fill_index = pl.pallas_call(
    _fill_index_body,
    out_shape=jax.ShapeDtypeStruct((B, N), jnp.int32),
)


@functools.cache
def _make_sc_gather():
    mesh = plsc.VectorSubcoreMesh(core_axis_name="c", subcore_axis_name="s")

    @functools.partial(
        pl.kernel,
        mesh=mesh,
        out_type=jax.ShapeDtypeStruct((ROWS, D), jnp.float32),
        scratch_types=[
            pltpu.VMEM((NCHUNK, CHUNK), jnp.int32),
            pltpu.VMEM((CHUNK, D), jnp.float32),
            pltpu.VMEM((CHUNK, D), jnp.float32),
            pltpu.VMEM((CHUNK, D), jnp.float32),
            pltpu.SemaphoreType.DMA,
            pltpu.SemaphoreType.DMA,
            pltpu.SemaphoreType.DMA,
            pltpu.SemaphoreType.DMA,
            pltpu.SemaphoreType.DMA,
            pltpu.SemaphoreType.DMA,
        ],
    )
    def sc_gather(x_hbm, idx_hbm, out_hbm, idx_v,
                  b0, b1, b2, g0, g1, g2, w0, w1, w2):
        NB = 3
        w = lax.axis_index("s") * 2 + lax.axis_index("c")
        pltpu.sync_copy(idx_hbm.at[w], idx_v)
        bufs = (b0, b1, b2)
        gsems = (g0, g1, g2)
        wsems = (w0, w1, w2)
        gcp = [None] * NCHUNK
        wcp = [None] * NCHUNK
        for c in range(NB):
            gcp[c] = pltpu.async_copy(x_hbm.at[idx_v.at[c]], bufs[c], gsems[c])
        for c in range(NCHUNK):
            b = c % NB
            gcp[c].wait()
            base = (w * NCHUNK + c) * CHUNK
            wcp[c] = pltpu.async_copy(
                bufs[b], out_hbm.at[pl.ds(base, CHUNK)], wsems[b])
            nxt = c + NB
            if nxt < NCHUNK:
                wcp[c].wait()
                gcp[nxt] = pltpu.async_copy(
                    x_hbm.at[idx_v.at[nxt]], bufs[b], gsems[b])
        for c in range(NCHUNK - NB, NCHUNK):
            wcp[c].wait()

    return sc_gather


def kernel(input, mask):
    gidx = _fill_index(mask)
    gidx3 = gidx.reshape(NW, NCHUNK, CHUNK)
    x2d = input.reshape(ROWS, D)
    out = _make_sc_gather()(x2d, gidx3)
    return out.reshape(B, N, D)
